# baseline (speedup 1.0000x reference)
import jax
import jax.numpy as jnp
from jax.experimental import pallas as pl
from jax.experimental.pallas import tpu as pltpu

N = 10000
D = 128
K = 32
R = 200
NP = 10112
ITERS_M = 31
ITERS = 26


def _ln(x, g, b, eps=1e-5):
    m = jnp.mean(x, axis=-1, keepdims=True)
    v = jnp.mean(jnp.square(x - m), axis=-1, keepdims=True)
    return (x - m) * jax.lax.rsqrt(v + eps) * g + b


def _body(nv1, nv2, src, tgt, w1, b1, w2, b2, g1, be1, g2, be2, out, adj_ref):
    adj = jax.lax.dot_general(
        nv1[...], nv2[...], (((1,), (1,)), ((), ())),
        preferred_element_type=jnp.float32)
    adj_ref[...] = adj

    cm = jnp.max(adj_ref[...].reshape(R, NP // 128, 128), axis=2)
    rowmax = jnp.max(cm, axis=1, keepdims=True)

    lo = jnp.zeros((R, 1), jnp.int32)
    hi = jax.lax.bitcast_convert_type(rowmax, jnp.int32)

    def step_m(_, carry):
        lo, hi = carry
        mid = lo + (hi - lo + 1) // 2
        midf = jax.lax.bitcast_convert_type(mid, jnp.float32)
        cnt = jnp.sum((cm >= midf).astype(jnp.float32),
                      axis=1, keepdims=True)
        ge = cnt >= K
        return jnp.where(ge, mid, lo), jnp.where(ge, hi, mid - 1)

    t0, _ = jax.lax.fori_loop(0, ITERS_M, step_m, (lo, hi))

    hi = jax.lax.bitcast_convert_type(rowmax, jnp.int32)

    def step(_, carry):
        lo, hi = carry
        mid = lo + (hi - lo + 1) // 2
        midf = jax.lax.bitcast_convert_type(mid, jnp.float32)
        cnt = jnp.sum((adj_ref[...] >= midf).astype(jnp.float32),
                      axis=1, keepdims=True)
        ge = cnt >= K
        return jnp.where(ge, mid, lo), jnp.where(ge, hi, mid - 1)

    lo, _ = jax.lax.fori_loop(0, ITERS, step, (t0, hi))
    thr = jax.lax.bitcast_convert_type(lo, jnp.float32)

    a = adj_ref[...]
    p = jnp.where(a >= thr, jnp.exp(a - rowmax), 0.0)
    z = jnp.sum(p, axis=1, keepdims=True)
    gc = jax.lax.dot_general(
        p, src[...], (((1,), (0,)), ((), ())),
        preferred_element_type=jnp.float32) / z

    t = _ln(tgt[...] + gc, g1[...], be1[...])
    h = jnp.maximum(
        jnp.dot(t, w1[...], preferred_element_type=jnp.float32) + b1[...], 0.0)
    ff = jnp.dot(h, w2[...], preferred_element_type=jnp.float32) + b2[...]
    out[...] = _ln(t + ff, g2[...], be2[...])


@jax.jit
def kernel(src, tgt, nodevec1, nodevec2, w1, b1, w2, b2, g1, be1, g2, be2):
    row = lambda v: v.reshape(1, D)
    pad = jnp.zeros((NP - N, D), jnp.float32)
    nv2p = jnp.concatenate([nodevec2, pad], axis=0)
    srcp = jnp.concatenate([src, pad], axis=0)
    full = pl.BlockSpec((NP, D), lambda i: (0, 0))
    blk = pl.BlockSpec((R, D), lambda i: (i, 0))
    vec = pl.BlockSpec((1, D), lambda i: (0, 0))
    mat = pl.BlockSpec((D, D), lambda i: (0, 0))
    return pl.pallas_call(
        _body,
        grid=(N // R,),
        in_specs=[blk, full, full, blk, mat, vec, mat, vec,
                  vec, vec, vec, vec],
        out_specs=blk,
        out_shape=jax.ShapeDtypeStruct((N, D), jnp.float32),
        scratch_shapes=[pltpu.VMEM((R, NP), jnp.float32)],
        compiler_params=pltpu.CompilerParams(
            dimension_semantics=("parallel",)),
    )(nodevec1, nv2p, srcp, tgt, w1, row(b1), w2, row(b2),
      row(g1), row(be1), row(g2), row(be2))
